# baseline (speedup 1.0000x reference)
import jax
import jax.numpy as jnp
from jax.experimental import pallas as pl
from jax.experimental.pallas import tpu as pltpu

IB = 32
BL = 128


def _even_cols_selector(pw, ow):
    r = jax.lax.broadcasted_iota(jnp.int32, (pw, ow), 0)
    c = jax.lax.broadcasted_iota(jnp.int32, (pw, ow), 1)
    return (c == 2 * r).astype(jnp.float32)


def _conv5x5(xpad, xcat, w_ref, *, cin, wp8, m_out):
    r_cat = xcat.shape[0]
    for j in range(5):
        xcat[:, j * cin:(j + 1) * cin] = xpad[pl.ds(j, r_cat), :]
    acc = None
    for i in range(5):
        part = jnp.dot(xcat[pl.ds(i * wp8, m_out), :], w_ref[i],
                       preferred_element_type=jnp.float32)
        acc = part if acc is None else acc + part
    return acc


def _pool_store(acc, b_ref, dst, *, wp8, ow, ph, pw, row0, stride, sel):
    bias = b_ref[...]
    for py in range(ph):
        r0 = 2 * py * wp8
        m = jnp.maximum(
            jnp.maximum(acc[pl.ds(r0, ow), :], acc[pl.ds(r0 + 1, ow), :]),
            jnp.maximum(acc[pl.ds(r0 + wp8, ow), :],
                        acc[pl.ds(r0 + wp8 + 1, ow), :]))
        p = jnp.dot(sel, m, preferred_element_type=jnp.float32)
        p = jnp.maximum(p + bias, 0.0).astype(dst.dtype)
        dst[pl.ds(row0 + py * stride, pw), :] = p


def _head_pipeline(i, x_ref, w1, b1, w2, b2, out_ref, sel1, sel2, bufs):
    xpad1, xcat1, acc1, xpad2, xcat2, acc2 = bufs
    base = i * 1024
    for y in range(32):
        xpad1[pl.ds((2 + y) * 40 + 2, 32), :] = \
            x_ref[pl.ds(base + y * 32, 32), :]

    acc1[...] = _conv5x5(xpad1, xcat1, w1, cin=8, wp8=40, m_out=1280)
    _pool_store(acc1, b1, xpad2, wp8=40, ow=32, ph=16, pw=16,
                row0=2 * 24 + 2, stride=24, sel=sel1)

    acc2[...] = _conv5x5(xpad2, xcat2, w2, cin=128, wp8=24, m_out=384)
    _pool_store(acc2, b2, out_ref, wp8=24, ow=16, ph=8, pw=8,
                row0=i * 64, stride=8, sel=sel2)


def _head_kernel(x_ref, w1, b1, w2, b2, out_ref, *scratch):
    bufs_a = scratch[:6]
    bufs_b = scratch[6:]
    for bufs in (bufs_a, bufs_b):
        bufs[0][...] = jnp.zeros_like(bufs[0])
        bufs[3][...] = jnp.zeros_like(bufs[3])

    sel1 = _even_cols_selector(16, 32)
    sel2 = _even_cols_selector(8, 16)

    def two_images(j, carry):
        _head_pipeline(2 * j, x_ref, w1, b1, w2, b2, out_ref,
                       sel1, sel2, bufs_a)
        _head_pipeline(2 * j + 1, x_ref, w1, b1, w2, b2, out_ref,
                       sel1, sel2, bufs_b)
        return carry

    jax.lax.fori_loop(0, IB // 2, two_images, 0, unroll=False)


def _cb_conv(w_ref, b_ref, src, dst, *, cin, cout, iw, oh, ow, dw, store):
    bias = b_ref[...]
    for y in range(oh):
        for x in range(ow):
            acc = None
            for kh in range(5):
                r0 = ((y + kh) * iw + x) * cin
                part = jnp.dot(w_ref[kh], src[pl.ds(r0, 5 * cin), :],
                               preferred_element_type=jnp.float32)
                acc = part if acc is None else acc + part
            v = jnp.maximum(acc + bias, 0.0)
            store(y * ow + x, v)
    _ = dw


def _tail_kernel(x_ref, w3, b3, w4, b4, w5, b5,
                 wl1, bl1, wl2, bl2, wl3, bl3, out_ref,
                 xp3, xp4, xp5, feat):
    @pl.when(pl.program_id(1) == 0)
    def _zero():
        xp3[...] = jnp.zeros_like(xp3)
        xp4[...] = jnp.zeros_like(xp4)
        xp5[...] = jnp.zeros_like(xp5)

    for pos in range(64):
        y, x = divmod(pos, 8)
        v = x_ref[:, pos, :]
        r = ((y + 1) * 10 + (x + 1)) * 128
        xp3[pl.ds(r, 128), :] = jnp.swapaxes(v, 0, 1)

    def store4(pos, v):
        y, x = divmod(pos, 6)
        r = ((y + 1) * 8 + (x + 1)) * 256
        xp4[pl.ds(r, 256), :] = v.astype(jnp.bfloat16)

    _cb_conv(w3, b3, xp3, xp4, cin=128, cout=256, iw=10, oh=6, ow=6,
             dw=8, store=store4)

    def store5(pos, v):
        y, x = divmod(pos, 4)
        r = ((y + 1) * 6 + (x + 1)) * 256
        xp5[pl.ds(r, 256), :] = v.astype(jnp.bfloat16)

    _cb_conv(w4, b4, xp4, xp5, cin=256, cout=256, iw=8, oh=4, ow=4,
             dw=6, store=store5)

    def storef(pos, v):
        feat[pl.ds(pos * 128, 128), :] = v.astype(jnp.bfloat16)

    _cb_conv(w5, b5, xp5, feat, cin=256, cout=128, iw=6, oh=2, ow=2,
             dw=0, store=storef)

    h = jnp.dot(wl1[...], feat[...],
                preferred_element_type=jnp.float32) + bl1[...]
    h = jnp.dot(wl2[...], h.astype(jnp.bfloat16),
                preferred_element_type=jnp.float32) + bl2[...]
    h = jnp.dot(wl3[...], h.astype(jnp.bfloat16),
                preferred_element_type=jnp.float32) + bl3[...]
    out_ref[...] = jnp.swapaxes(1.0 / (1.0 + jnp.exp(-h)), 0, 1)


def _whole(shape):
    return pl.BlockSpec(shape, lambda *g: tuple(0 for _ in shape))


def kernel(x, c1_w, c1_b, c2_w, c2_b, c3_w, c3_b, c4_w, c4_b, c5_w, c5_b,
           l1_w, l1_b, l2_w, l2_b, l3_w, l3_b):
    n = x.shape[0]
    xh = jnp.transpose(x, (0, 2, 3, 1)).astype(jnp.bfloat16)
    xh = jnp.pad(xh, ((0, 0), (0, 0), (0, 0), (0, 5)))
    x2d = xh.reshape(n * 1024, 8)

    pooled = pl.pallas_call(
        _head_kernel,
        out_shape=jax.ShapeDtypeStruct((n * 64, 128), jnp.bfloat16),
        grid=(n // IB,),
        in_specs=[
            pl.BlockSpec((IB * 1024, 8), lambda g: (g, 0)),
            _whole((5, 40, 128)), _whole((1, 128)),
            _whole((5, 640, 128)), _whole((1, 128)),
        ],
        out_specs=pl.BlockSpec((IB * 64, 128), lambda g: (g, 0)),
        scratch_shapes=[
            pltpu.VMEM((1448, 8), jnp.bfloat16),
            pltpu.VMEM((1440, 40), jnp.bfloat16),
            pltpu.VMEM((1280, 128), jnp.float32),
            pltpu.VMEM((488, 128), jnp.bfloat16),
            pltpu.VMEM((480, 640), jnp.bfloat16),
            pltpu.VMEM((384, 128), jnp.float32),
        ] * 2,
        compiler_params=pltpu.CompilerParams(
            dimension_semantics=("parallel",),
            vmem_limit_bytes=64 * 1024 * 1024,
        ),
    )(x2d, c1_w, c1_b, c2_w, c2_b)

    nb = n // BL
    w3t = jnp.swapaxes(c3_w, 1, 2)
    w4t = jnp.swapaxes(c4_w, 1, 2)
    w5t = jnp.swapaxes(c5_w, 1, 2)

    out = pl.pallas_call(
        _tail_kernel,
        out_shape=jax.ShapeDtypeStruct((n, 128), jnp.float32),
        grid=(2, nb // 2),
        in_specs=[
            pl.BlockSpec((BL, 64, 128),
                         lambda c, j: (c * (nb // 2) + j, 0, 0)),
            _whole((5, 256, 640)), _whole((256, 1)),
            _whole((5, 256, 1280)), _whole((256, 1)),
            _whole((5, 128, 1280)), _whole((128, 1)),
            _whole((384, 512)), _whole((384, 1)),
            _whole((256, 384)), _whole((256, 1)),
            _whole((128, 256)), _whole((128, 1)),
        ],
        out_specs=pl.BlockSpec((BL, 128),
                               lambda c, j: (c * (nb // 2) + j, 0)),
        scratch_shapes=[
            pltpu.VMEM((10 * 10 * 128, BL), jnp.bfloat16),
            pltpu.VMEM((8 * 8 * 256, BL), jnp.bfloat16),
            pltpu.VMEM((6 * 6 * 256, BL), jnp.bfloat16),
            pltpu.VMEM((512, BL), jnp.bfloat16),
        ],
        compiler_params=pltpu.CompilerParams(
            dimension_semantics=("parallel", "arbitrary"),
            vmem_limit_bytes=64 * 1024 * 1024,
        ),
    )(pooled.reshape(n, 64, 128), w3t, c3_b.reshape(256, 1),
      w4t, c4_b.reshape(256, 1),
      w5t, c5_b.reshape(128, 1), l1_w.T, l1_b.reshape(384, 1),
      l2_w.T, l2_b.reshape(256, 1), l3_w.T, l3_b.reshape(128, 1))
    return out[:, :100]


# speedup vs baseline: 1.3076x; 1.3076x over previous
import jax
import jax.numpy as jnp
from jax.experimental import pallas as pl
from jax.experimental.pallas import tpu as pltpu

IB = 32
BL = 128


def _even_cols_selector(pw, ow):
    r = jax.lax.broadcasted_iota(jnp.int32, (pw, ow), 0)
    c = jax.lax.broadcasted_iota(jnp.int32, (pw, ow), 1)
    return (c == 2 * r).astype(jnp.float32)


def _head_pipeline(i, x_ref, w1, b1, out_ref, sel1, bufs):
    xpad1, xcat1, acc1 = bufs
    base = i * 1024
    for y in range(32):
        xpad1[pl.ds((2 + y) * 40 + 2, 32), :] = \
            x_ref[pl.ds(base + y * 32, 32), :]

    for j in range(5):
        xcat1[:, j * 8:(j + 1) * 8] = xpad1[pl.ds(j, 1440), :]
    acc = None
    for t in range(5):
        part = jnp.dot(xcat1[pl.ds(t * 40, 1280), :], w1[t],
                       preferred_element_type=jnp.float32)
        acc = part if acc is None else acc + part
    acc1[...] = acc

    bias = b1[...]
    for py in range(16):
        r0 = 2 * py * 40
        m = jnp.maximum(
            jnp.maximum(acc1[pl.ds(r0, 32), :], acc1[pl.ds(r0 + 1, 32), :]),
            jnp.maximum(acc1[pl.ds(r0 + 40, 32), :],
                        acc1[pl.ds(r0 + 41, 32), :]))
        p = jnp.dot(sel1, m, preferred_element_type=jnp.float32)
        p = jnp.maximum(p + bias, 0.0).astype(out_ref.dtype)
        out_ref[pl.ds(i * 256 + py * 16, 16), :] = p


def _head_kernel(x_ref, w1, b1, out_ref, *scratch):
    bufs_a = scratch[:3]
    bufs_b = scratch[3:]
    for bufs in (bufs_a, bufs_b):
        bufs[0][...] = jnp.zeros_like(bufs[0])

    sel1 = _even_cols_selector(16, 32)

    def two_images(j, carry):
        _head_pipeline(2 * j, x_ref, w1, b1, out_ref, sel1, bufs_a)
        _head_pipeline(2 * j + 1, x_ref, w1, b1, out_ref, sel1, bufs_b)
        return carry

    jax.lax.fori_loop(0, IB // 2, two_images, 0, unroll=False)


def _cb_tap_sum(w_ref, src, *, cin, iw, y, x):
    acc = None
    for kh in range(5):
        r0 = ((y + kh) * iw + x) * cin
        part = jnp.dot(w_ref[kh], src[pl.ds(r0, 5 * cin), :],
                       preferred_element_type=jnp.float32)
        acc = part if acc is None else acc + part
    return acc


def _cb_conv(w_ref, b_ref, src, *, cin, iw, oh, ow, store):
    bias = b_ref[...]
    for y in range(oh):
        for x in range(ow):
            v = _cb_tap_sum(w_ref, src, cin=cin, iw=iw, y=y, x=x)
            store(y * ow + x, jnp.maximum(v + bias, 0.0))


def _tail_kernel(x_ref, w2, b2, w3, b3, w4, b4, w5, b5,
                 wl1, bl1, wl2, bl2, wl3, bl3, out_ref,
                 xp2, xp3, xp4, xp5, feat):
    @pl.when(pl.program_id(1) == 0)
    def _zero():
        xp2[...] = jnp.zeros_like(xp2)
        xp3[...] = jnp.zeros_like(xp3)
        xp4[...] = jnp.zeros_like(xp4)
        xp5[...] = jnp.zeros_like(xp5)

    for pos in range(256):
        y, x = divmod(pos, 16)
        v = x_ref[:, pos, :]
        r = ((y + 2) * 20 + (x + 2)) * 128
        xp2[pl.ds(r, 128), :] = jnp.swapaxes(v, 0, 1)

    b2v = b2[...]
    for py in range(8):
        for px in range(8):
            cell = None
            for dy in range(2):
                for dx in range(2):
                    v = _cb_tap_sum(w2, xp2, cin=128, iw=20,
                                    y=2 * py + dy, x=2 * px + dx)
                    cell = v if cell is None else jnp.maximum(cell, v)
            v = jnp.maximum(cell + b2v, 0.0)
            r = ((py + 1) * 10 + (px + 1)) * 128
            xp3[pl.ds(r, 128), :] = v.astype(jnp.bfloat16)

    def store4(pos, v):
        y, x = divmod(pos, 6)
        r = ((y + 1) * 8 + (x + 1)) * 256
        xp4[pl.ds(r, 256), :] = v.astype(jnp.bfloat16)

    _cb_conv(w3, b3, xp3, cin=128, iw=10, oh=6, ow=6, store=store4)

    def store5(pos, v):
        y, x = divmod(pos, 4)
        r = ((y + 1) * 6 + (x + 1)) * 256
        xp5[pl.ds(r, 256), :] = v.astype(jnp.bfloat16)

    _cb_conv(w4, b4, xp4, cin=256, iw=8, oh=4, ow=4, store=store5)

    def storef(pos, v):
        feat[pl.ds(pos * 128, 128), :] = v.astype(jnp.bfloat16)

    _cb_conv(w5, b5, xp5, cin=256, iw=6, oh=2, ow=2, store=storef)

    h = jnp.dot(wl1[...], feat[...],
                preferred_element_type=jnp.float32) + bl1[...]
    h = jnp.dot(wl2[...], h.astype(jnp.bfloat16),
                preferred_element_type=jnp.float32) + bl2[...]
    h = jnp.dot(wl3[...], h.astype(jnp.bfloat16),
                preferred_element_type=jnp.float32) + bl3[...]
    out_ref[...] = jnp.swapaxes(1.0 / (1.0 + jnp.exp(-h)), 0, 1)


def _whole(shape):
    return pl.BlockSpec(shape, lambda *g: tuple(0 for _ in shape))


def kernel(x, c1_w, c1_b, c2_w, c2_b, c3_w, c3_b, c4_w, c4_b, c5_w, c5_b,
           l1_w, l1_b, l2_w, l2_b, l3_w, l3_b):
    n = x.shape[0]
    xh = jnp.transpose(x, (0, 2, 3, 1)).astype(jnp.bfloat16)
    xh = jnp.pad(xh, ((0, 0), (0, 0), (0, 0), (0, 5)))
    x2d = xh.reshape(n * 1024, 8)

    pooled = pl.pallas_call(
        _head_kernel,
        out_shape=jax.ShapeDtypeStruct((n * 256, 128), jnp.bfloat16),
        grid=(n // IB,),
        in_specs=[
            pl.BlockSpec((IB * 1024, 8), lambda g: (g, 0)),
            _whole((5, 40, 128)), _whole((1, 128)),
        ],
        out_specs=pl.BlockSpec((IB * 256, 128), lambda g: (g, 0)),
        scratch_shapes=[
            pltpu.VMEM((1448, 8), jnp.bfloat16),
            pltpu.VMEM((1440, 40), jnp.bfloat16),
            pltpu.VMEM((1280, 128), jnp.float32),
        ] * 2,
        compiler_params=pltpu.CompilerParams(
            dimension_semantics=("parallel",),
            vmem_limit_bytes=64 * 1024 * 1024,
        ),
    )(x2d, c1_w, c1_b)

    nb = n // BL
    w2t = jnp.swapaxes(c2_w, 1, 2)
    w3t = jnp.swapaxes(c3_w, 1, 2)
    w4t = jnp.swapaxes(c4_w, 1, 2)
    w5t = jnp.swapaxes(c5_w, 1, 2)

    out = pl.pallas_call(
        _tail_kernel,
        out_shape=jax.ShapeDtypeStruct((n, 128), jnp.float32),
        grid=(2, nb // 2),
        in_specs=[
            pl.BlockSpec((BL, 256, 128),
                         lambda c, j: (c * (nb // 2) + j, 0, 0)),
            _whole((5, 128, 640)), _whole((128, 1)),
            _whole((5, 256, 640)), _whole((256, 1)),
            _whole((5, 256, 1280)), _whole((256, 1)),
            _whole((5, 128, 1280)), _whole((128, 1)),
            _whole((384, 512)), _whole((384, 1)),
            _whole((256, 384)), _whole((256, 1)),
            _whole((128, 256)), _whole((128, 1)),
        ],
        out_specs=pl.BlockSpec((BL, 128),
                               lambda c, j: (c * (nb // 2) + j, 0)),
        scratch_shapes=[
            pltpu.VMEM((20 * 20 * 128, BL), jnp.bfloat16),
            pltpu.VMEM((10 * 10 * 128, BL), jnp.bfloat16),
            pltpu.VMEM((8 * 8 * 256, BL), jnp.bfloat16),
            pltpu.VMEM((6 * 6 * 256, BL), jnp.bfloat16),
            pltpu.VMEM((512, BL), jnp.bfloat16),
        ],
        compiler_params=pltpu.CompilerParams(
            dimension_semantics=("parallel", "arbitrary"),
            vmem_limit_bytes=110 * 1024 * 1024,
        ),
    )(pooled.reshape(n, 256, 128), w2t, c2_b.reshape(128, 1),
      w3t, c3_b.reshape(256, 1), w4t, c4_b.reshape(256, 1),
      w5t, c5_b.reshape(128, 1), l1_w.T, l1_b.reshape(384, 1),
      l2_w.T, l2_b.reshape(256, 1), l3_w.T, l3_b.reshape(128, 1))
    return out[:, :100]


# speedup vs baseline: 1.4195x; 1.0856x over previous
import jax
import jax.numpy as jnp
from jax.experimental import pallas as pl
from jax.experimental.pallas import tpu as pltpu

IB = 32
BL = 128


def _even_cols_selector(pw, ow):
    r = jax.lax.broadcasted_iota(jnp.int32, (pw, ow), 0)
    c = jax.lax.broadcasted_iota(jnp.int32, (pw, ow), 1)
    return (c == 2 * r).astype(jnp.float32)


def _head_pipeline(i, x_ref, w1, b1, out_ref, sel1, bufs):
    xpad1, xcat1, acc1 = bufs
    base = i * 1024
    for y in range(32):
        xpad1[pl.ds((2 + y) * 40 + 2, 32), :] = \
            x_ref[pl.ds(base + y * 32, 32), :]

    for j in range(5):
        xcat1[:, j * 8:(j + 1) * 8] = xpad1[pl.ds(j, 1440), :]
    acc = None
    for t in range(5):
        part = jnp.dot(xcat1[pl.ds(t * 40, 1280), :], w1[t],
                       preferred_element_type=jnp.float32)
        acc = part if acc is None else acc + part
    acc1[...] = acc

    bias = b1[...]
    for py in range(16):
        r0 = 2 * py * 40
        m = jnp.maximum(
            jnp.maximum(acc1[pl.ds(r0, 32), :], acc1[pl.ds(r0 + 1, 32), :]),
            jnp.maximum(acc1[pl.ds(r0 + 40, 32), :],
                        acc1[pl.ds(r0 + 41, 32), :]))
        p = jnp.dot(sel1, m, preferred_element_type=jnp.float32)
        p = jnp.maximum(p + bias, 0.0).astype(out_ref.dtype)
        out_ref[pl.ds(i * 256 + py * 16, 16), :] = p


def _head_kernel(x_ref, w1, b1, out_ref, *scratch):
    sets = [scratch[3 * k:3 * k + 3] for k in range(4)]
    for bufs in sets:
        bufs[0][...] = jnp.zeros_like(bufs[0])

    sel1 = _even_cols_selector(16, 32)

    def four_images(j, carry):
        for k, bufs in enumerate(sets):
            _head_pipeline(4 * j + k, x_ref, w1, b1, out_ref, sel1, bufs)
        return carry

    jax.lax.fori_loop(0, IB // 4, four_images, 0, unroll=False)


def _cb_tap_sum(w_ref, src, *, cin, iw, y, x):
    acc = None
    for kh in range(5):
        r0 = ((y + kh) * iw + x) * cin
        part = jnp.dot(w_ref[kh], src[pl.ds(r0, 5 * cin), :],
                       preferred_element_type=jnp.float32)
        acc = part if acc is None else acc + part
    return acc


def _cb_conv(w_ref, b_ref, src, *, cin, iw, oh, ow, store):
    bias = b_ref[...]
    for y in range(oh):
        for x in range(ow):
            v = _cb_tap_sum(w_ref, src, cin=cin, iw=iw, y=y, x=x)
            store(y * ow + x, jnp.maximum(v + bias, 0.0))


def _tail_kernel(x_ref, w2, b2, w3, b3, w4, b4, w5, b5,
                 wl1, bl1, wl2, bl2, wl3, bl3, out_ref,
                 xp2, xp3, xp4, xp5, feat):
    @pl.when(pl.program_id(1) == 0)
    def _zero():
        xp2[...] = jnp.zeros_like(xp2)
        xp3[...] = jnp.zeros_like(xp3)
        xp4[...] = jnp.zeros_like(xp4)
        xp5[...] = jnp.zeros_like(xp5)

    for pos in range(256):
        y, x = divmod(pos, 16)
        v = x_ref[:, pos, :]
        r = ((y + 2) * 20 + (x + 2)) * 128
        xp2[pl.ds(r, 128), :] = jnp.swapaxes(v, 0, 1)

    b2v = b2[...]
    for py in range(8):
        for px in range(8):
            cell = None
            for dy in range(2):
                for dx in range(2):
                    v = _cb_tap_sum(w2, xp2, cin=128, iw=20,
                                    y=2 * py + dy, x=2 * px + dx)
                    cell = v if cell is None else jnp.maximum(cell, v)
            v = jnp.maximum(cell + b2v, 0.0)
            r = ((py + 1) * 10 + (px + 1)) * 128
            xp3[pl.ds(r, 128), :] = v.astype(jnp.bfloat16)

    def store4(pos, v):
        y, x = divmod(pos, 6)
        r = ((y + 1) * 8 + (x + 1)) * 256
        xp4[pl.ds(r, 256), :] = v.astype(jnp.bfloat16)

    _cb_conv(w3, b3, xp3, cin=128, iw=10, oh=6, ow=6, store=store4)

    def store5(pos, v):
        y, x = divmod(pos, 4)
        r = ((y + 1) * 6 + (x + 1)) * 256
        xp5[pl.ds(r, 256), :] = v.astype(jnp.bfloat16)

    _cb_conv(w4, b4, xp4, cin=256, iw=8, oh=4, ow=4, store=store5)

    def storef(pos, v):
        feat[pl.ds(pos * 128, 128), :] = v.astype(jnp.bfloat16)

    _cb_conv(w5, b5, xp5, cin=256, iw=6, oh=2, ow=2, store=storef)

    h = jnp.dot(wl1[...], feat[...],
                preferred_element_type=jnp.float32) + bl1[...]
    h = jnp.dot(wl2[...], h.astype(jnp.bfloat16),
                preferred_element_type=jnp.float32) + bl2[...]
    h = jnp.dot(wl3[...], h.astype(jnp.bfloat16),
                preferred_element_type=jnp.float32) + bl3[...]
    out_ref[...] = jnp.swapaxes(1.0 / (1.0 + jnp.exp(-h)), 0, 1)


def _whole(shape):
    return pl.BlockSpec(shape, lambda *g: tuple(0 for _ in shape))


def kernel(x, c1_w, c1_b, c2_w, c2_b, c3_w, c3_b, c4_w, c4_b, c5_w, c5_b,
           l1_w, l1_b, l2_w, l2_b, l3_w, l3_b):
    n = x.shape[0]
    xh = jnp.transpose(x, (0, 2, 3, 1)).astype(jnp.bfloat16)
    xh = jnp.pad(xh, ((0, 0), (0, 0), (0, 0), (0, 5)))
    x2d = xh.reshape(n * 1024, 8)

    pooled = pl.pallas_call(
        _head_kernel,
        out_shape=jax.ShapeDtypeStruct((n * 256, 128), jnp.bfloat16),
        grid=(n // IB,),
        in_specs=[
            pl.BlockSpec((IB * 1024, 8), lambda g: (g, 0)),
            _whole((5, 40, 128)), _whole((1, 128)),
        ],
        out_specs=pl.BlockSpec((IB * 256, 128), lambda g: (g, 0)),
        scratch_shapes=[
            pltpu.VMEM((1448, 8), jnp.bfloat16),
            pltpu.VMEM((1440, 40), jnp.bfloat16),
            pltpu.VMEM((1280, 128), jnp.float32),
        ] * 4,
        compiler_params=pltpu.CompilerParams(
            dimension_semantics=("parallel",),
            vmem_limit_bytes=64 * 1024 * 1024,
        ),
    )(x2d, c1_w, c1_b)

    nb = n // BL
    w2t = jnp.swapaxes(c2_w, 1, 2)
    w3t = jnp.swapaxes(c3_w, 1, 2)
    w4t = jnp.swapaxes(c4_w, 1, 2)
    w5t = jnp.swapaxes(c5_w, 1, 2)

    out = pl.pallas_call(
        _tail_kernel,
        out_shape=jax.ShapeDtypeStruct((n, 128), jnp.float32),
        grid=(2, nb // 2),
        in_specs=[
            pl.BlockSpec((BL, 256, 128),
                         lambda c, j: (c * (nb // 2) + j, 0, 0)),
            _whole((5, 128, 640)), _whole((128, 1)),
            _whole((5, 256, 640)), _whole((256, 1)),
            _whole((5, 256, 1280)), _whole((256, 1)),
            _whole((5, 128, 1280)), _whole((128, 1)),
            _whole((384, 512)), _whole((384, 1)),
            _whole((256, 384)), _whole((256, 1)),
            _whole((128, 256)), _whole((128, 1)),
        ],
        out_specs=pl.BlockSpec((BL, 128),
                               lambda c, j: (c * (nb // 2) + j, 0)),
        scratch_shapes=[
            pltpu.VMEM((20 * 20 * 128, BL), jnp.bfloat16),
            pltpu.VMEM((10 * 10 * 128, BL), jnp.bfloat16),
            pltpu.VMEM((8 * 8 * 256, BL), jnp.bfloat16),
            pltpu.VMEM((6 * 6 * 256, BL), jnp.bfloat16),
            pltpu.VMEM((512, BL), jnp.bfloat16),
        ],
        compiler_params=pltpu.CompilerParams(
            dimension_semantics=("parallel", "arbitrary"),
            vmem_limit_bytes=110 * 1024 * 1024,
        ),
    )(pooled.reshape(n, 256, 128), w2t, c2_b.reshape(128, 1),
      w3t, c3_b.reshape(256, 1), w4t, c4_b.reshape(256, 1),
      w5t, c5_b.reshape(128, 1), l1_w.T, l1_b.reshape(384, 1),
      l2_w.T, l2_b.reshape(256, 1), l3_w.T, l3_b.reshape(128, 1))
    return out[:, :100]


# speedup vs baseline: 1.4583x; 1.0273x over previous
import jax
import jax.numpy as jnp
from jax.experimental import pallas as pl
from jax.experimental.pallas import tpu as pltpu

IB = 32
BL = 128


def _even_cols_selector(pw, ow):
    r = jax.lax.broadcasted_iota(jnp.int32, (pw, ow), 0)
    c = jax.lax.broadcasted_iota(jnp.int32, (pw, ow), 1)
    return (c == 2 * r).astype(jnp.float32)


def _head_pipeline(i, x_ref, w1, b1, out_ref, sel1, bufs):
    xpad1, xcat1, acc1 = bufs
    base = i * 1024
    for y in range(32):
        xpad1[pl.ds((2 + y) * 40 + 2, 32), :] = \
            x_ref[pl.ds(base + y * 32, 32), :]

    for j in range(5):
        xcat1[:, j * 8:(j + 1) * 8] = xpad1[pl.ds(j, 1440), :]
    acc = None
    for t in range(5):
        part = jnp.dot(xcat1[pl.ds(t * 40, 1280), :], w1[t],
                       preferred_element_type=jnp.float32)
        acc = part if acc is None else acc + part
    acc1[...] = acc

    bias = b1[...]
    for py in range(16):
        r0 = 2 * py * 40
        m = jnp.maximum(
            jnp.maximum(acc1[pl.ds(r0, 32), :], acc1[pl.ds(r0 + 1, 32), :]),
            jnp.maximum(acc1[pl.ds(r0 + 40, 32), :],
                        acc1[pl.ds(r0 + 41, 32), :]))
        p = jnp.dot(sel1, m, preferred_element_type=jnp.float32)
        p = jnp.maximum(p + bias, 0.0).astype(out_ref.dtype)
        out_ref[pl.ds(i * 256 + py * 16, 16), :] = p


def _head_kernel(x_ref, w1, b1, out_ref, *scratch):
    sets = [scratch[3 * k:3 * k + 3] for k in range(8)]
    for bufs in sets:
        bufs[0][...] = jnp.zeros_like(bufs[0])

    sel1 = _even_cols_selector(16, 32)

    def eight_images(j, carry):
        for k, bufs in enumerate(sets):
            _head_pipeline(8 * j + k, x_ref, w1, b1, out_ref, sel1, bufs)
        return carry

    jax.lax.fori_loop(0, IB // 8, eight_images, 0, unroll=False)


def _cb_tap_sum(w_ref, src, *, cin, iw, y, x):
    acc = None
    for kh in range(5):
        r0 = ((y + kh) * iw + x) * cin
        part = jnp.dot(w_ref[kh], src[pl.ds(r0, 5 * cin), :],
                       preferred_element_type=jnp.float32)
        acc = part if acc is None else acc + part
    return acc


def _cb_conv(w_ref, b_ref, src, *, cin, iw, oh, ow, store):
    bias = b_ref[...]
    for y in range(oh):
        for x in range(ow):
            v = _cb_tap_sum(w_ref, src, cin=cin, iw=iw, y=y, x=x)
            store(y * ow + x, jnp.maximum(v + bias, 0.0))


def _tail_kernel(x_ref, w2, b2, w3, b3, w4, b4, w5, b5,
                 wl1, bl1, wl2, bl2, wl3, bl3, out_ref,
                 xp2, xp3, xp4, xp5, feat):
    @pl.when(pl.program_id(1) == 0)
    def _zero():
        xp2[...] = jnp.zeros_like(xp2)
        xp3[...] = jnp.zeros_like(xp3)
        xp4[...] = jnp.zeros_like(xp4)
        xp5[...] = jnp.zeros_like(xp5)

    for pos in range(256):
        y, x = divmod(pos, 16)
        v = x_ref[:, pos, :]
        r = ((y + 2) * 20 + (x + 2)) * 128
        xp2[pl.ds(r, 128), :] = jnp.swapaxes(v, 0, 1)

    b2v = b2[...]
    for py in range(8):
        for px in range(8):
            cell = None
            for dy in range(2):
                for dx in range(2):
                    v = _cb_tap_sum(w2, xp2, cin=128, iw=20,
                                    y=2 * py + dy, x=2 * px + dx)
                    cell = v if cell is None else jnp.maximum(cell, v)
            v = jnp.maximum(cell + b2v, 0.0)
            r = ((py + 1) * 10 + (px + 1)) * 128
            xp3[pl.ds(r, 128), :] = v.astype(jnp.bfloat16)

    def store4(pos, v):
        y, x = divmod(pos, 6)
        r = ((y + 1) * 8 + (x + 1)) * 256
        xp4[pl.ds(r, 256), :] = v.astype(jnp.bfloat16)

    _cb_conv(w3, b3, xp3, cin=128, iw=10, oh=6, ow=6, store=store4)

    def store5(pos, v):
        y, x = divmod(pos, 4)
        r = ((y + 1) * 6 + (x + 1)) * 256
        xp5[pl.ds(r, 256), :] = v.astype(jnp.bfloat16)

    _cb_conv(w4, b4, xp4, cin=256, iw=8, oh=4, ow=4, store=store5)

    def storef(pos, v):
        feat[pl.ds(pos * 128, 128), :] = v.astype(jnp.bfloat16)

    _cb_conv(w5, b5, xp5, cin=256, iw=6, oh=2, ow=2, store=storef)

    h = jnp.dot(wl1[...], feat[...],
                preferred_element_type=jnp.float32) + bl1[...]
    h = jnp.dot(wl2[...], h.astype(jnp.bfloat16),
                preferred_element_type=jnp.float32) + bl2[...]
    h = jnp.dot(wl3[...], h.astype(jnp.bfloat16),
                preferred_element_type=jnp.float32) + bl3[...]
    out_ref[...] = jnp.swapaxes(1.0 / (1.0 + jnp.exp(-h)), 0, 1)


def _whole(shape):
    return pl.BlockSpec(shape, lambda *g: tuple(0 for _ in shape))


def kernel(x, c1_w, c1_b, c2_w, c2_b, c3_w, c3_b, c4_w, c4_b, c5_w, c5_b,
           l1_w, l1_b, l2_w, l2_b, l3_w, l3_b):
    n = x.shape[0]
    xh = jnp.transpose(x, (0, 2, 3, 1)).astype(jnp.bfloat16)
    xh = jnp.pad(xh, ((0, 0), (0, 0), (0, 0), (0, 5)))
    x2d = xh.reshape(n * 1024, 8)

    pooled = pl.pallas_call(
        _head_kernel,
        out_shape=jax.ShapeDtypeStruct((n * 256, 128), jnp.bfloat16),
        grid=(n // IB,),
        in_specs=[
            pl.BlockSpec((IB * 1024, 8), lambda g: (g, 0)),
            _whole((5, 40, 128)), _whole((1, 128)),
        ],
        out_specs=pl.BlockSpec((IB * 256, 128), lambda g: (g, 0)),
        scratch_shapes=[
            pltpu.VMEM((1448, 8), jnp.bfloat16),
            pltpu.VMEM((1440, 40), jnp.bfloat16),
            pltpu.VMEM((1280, 128), jnp.float32),
        ] * 8,
        compiler_params=pltpu.CompilerParams(
            dimension_semantics=("parallel",),
            vmem_limit_bytes=64 * 1024 * 1024,
        ),
    )(x2d, c1_w, c1_b)

    nb = n // BL
    w2t = jnp.swapaxes(c2_w, 1, 2)
    w3t = jnp.swapaxes(c3_w, 1, 2)
    w4t = jnp.swapaxes(c4_w, 1, 2)
    w5t = jnp.swapaxes(c5_w, 1, 2)

    out = pl.pallas_call(
        _tail_kernel,
        out_shape=jax.ShapeDtypeStruct((n, 128), jnp.float32),
        grid=(2, nb // 2),
        in_specs=[
            pl.BlockSpec((BL, 256, 128),
                         lambda c, j: (c * (nb // 2) + j, 0, 0)),
            _whole((5, 128, 640)), _whole((128, 1)),
            _whole((5, 256, 640)), _whole((256, 1)),
            _whole((5, 256, 1280)), _whole((256, 1)),
            _whole((5, 128, 1280)), _whole((128, 1)),
            _whole((384, 512)), _whole((384, 1)),
            _whole((256, 384)), _whole((256, 1)),
            _whole((128, 256)), _whole((128, 1)),
        ],
        out_specs=pl.BlockSpec((BL, 128),
                               lambda c, j: (c * (nb // 2) + j, 0)),
        scratch_shapes=[
            pltpu.VMEM((20 * 20 * 128, BL), jnp.bfloat16),
            pltpu.VMEM((10 * 10 * 128, BL), jnp.bfloat16),
            pltpu.VMEM((8 * 8 * 256, BL), jnp.bfloat16),
            pltpu.VMEM((6 * 6 * 256, BL), jnp.bfloat16),
            pltpu.VMEM((512, BL), jnp.bfloat16),
        ],
        compiler_params=pltpu.CompilerParams(
            dimension_semantics=("parallel", "arbitrary"),
            vmem_limit_bytes=110 * 1024 * 1024,
        ),
    )(pooled.reshape(n, 256, 128), w2t, c2_b.reshape(128, 1),
      w3t, c3_b.reshape(256, 1), w4t, c4_b.reshape(256, 1),
      w5t, c5_b.reshape(128, 1), l1_w.T, l1_b.reshape(384, 1),
      l2_w.T, l2_b.reshape(256, 1), l3_w.T, l3_b.reshape(128, 1))
    return out[:, :100]
